# Initial kernel scaffold; baseline (speedup 1.0000x reference)
import jax
import jax.numpy as jnp
from jax.experimental import pallas as pl


def kernel(x, edge_index, Wl1, bl1, Wr1, Wl2, bl2, Wr2):
    raise NotImplementedError("write your pallas kernel here")


# speedup vs baseline: 7.0067x; 7.0067x over previous
import functools

import jax
import jax.numpy as jnp
from jax import lax
from jax.experimental import pallas as pl
from jax.experimental.pallas import tpu as pltpu
from jax.experimental.pallas import tpu_sc as plsc

N_NODES = 10000
N_EDGES = 320000
D = 128

NC = 2
NS = 16
NW = NC * NS
EPW = N_EDGES // NW
CHUNK = 80
NCHUNK = EPW // CHUNK
IO_TILES = 10
ROWS_PT = N_NODES // IO_TILES
CNT_PAD = 10240
CNT_PT = CNT_PAD // IO_TILES


def _segsum_body(x_hbm, src_hbm, dst_hbm, out_agg, out_cnt,
                 idx_s, idx_d, rows, ones_v, zbuf, zcbuf,
                 agg_sh, cnt_sh, sem):
    c = lax.axis_index("c")
    s = lax.axis_index("s")
    wid = s * NC + c

    zero16 = jnp.zeros((16,), jnp.float32)
    one16 = jnp.ones((16,), jnp.float32)

    def zrow(i, carry):
        for k in range(8):
            zbuf[i, pl.ds(16 * k, 16)] = zero16
        return carry
    lax.fori_loop(0, zbuf.shape[0], zrow, 0)

    def zc(i, carry):
        zcbuf[pl.ds(i * 16, 16)] = zero16
        return carry
    lax.fori_loop(0, zcbuf.shape[0] // 16, zc, 0)

    for k in range(CHUNK // 16):
        ones_v[pl.ds(16 * k, 16)] = one16

    @pl.when(s < IO_TILES)
    def _():
        def zcopy(k, carry):
            pltpu.sync_copy(zbuf, agg_sh.at[pl.ds(s * ROWS_PT + k * 40, 40)])
            return carry
        lax.fori_loop(0, ROWS_PT // 40, zcopy, 0)
        pltpu.sync_copy(zcbuf, cnt_sh.at[pl.ds(s * CNT_PT, CNT_PT)])

    plsc.subcore_barrier()

    pltpu.sync_copy(src_hbm.at[wid], idx_s)
    pltpu.sync_copy(dst_hbm.at[wid], idx_d)

    def step(j, carry):
        si = idx_s.at[j]
        di = idx_d.at[j]
        pltpu.async_copy(x_hbm.at[si], rows, sem).wait()
        pltpu.sync_copy(rows, agg_sh.at[di], add=True)
        pltpu.sync_copy(ones_v, cnt_sh.at[di], add=True)
        return carry
    lax.fori_loop(0, NCHUNK, step, 0)

    plsc.subcore_barrier()

    @pl.when(s < IO_TILES)
    def _():
        pltpu.sync_copy(agg_sh.at[pl.ds(s * ROWS_PT, ROWS_PT)],
                        out_agg.at[c, pl.ds(s * ROWS_PT, ROWS_PT)])
        pltpu.sync_copy(cnt_sh.at[pl.ds(s * CNT_PT, CNT_PT)],
                        out_cnt.at[pl.ds(c * CNT_PAD + s * CNT_PT, CNT_PT)])


_segsum = pl.kernel(
    _segsum_body,
    mesh=plsc.VectorSubcoreMesh(core_axis_name="c", subcore_axis_name="s"),
    out_type=[
        jax.ShapeDtypeStruct((NC, N_NODES, D), jnp.float32),
        jax.ShapeDtypeStruct((NC * CNT_PAD,), jnp.float32),
    ],
    scratch_types=[
        pltpu.VMEM((NCHUNK, CHUNK), jnp.int32),
        pltpu.VMEM((NCHUNK, CHUNK), jnp.int32),
        pltpu.VMEM((CHUNK, D), jnp.float32),
        pltpu.VMEM((CHUNK,), jnp.float32),
        pltpu.VMEM((40, D), jnp.float32),
        pltpu.VMEM((CNT_PT,), jnp.float32),
        pltpu.VMEM_SHARED((N_NODES, D), jnp.float32),
        pltpu.VMEM_SHARED((CNT_PAD,), jnp.float32),
        pltpu.SemaphoreType.DMA,
    ],
)


BN = 1000


def _tc1_body(aggp_ref, cntT_ref, x_ref, wlT_ref, wrT_ref, bl_ref, out_ref):
    a = aggp_ref[0] + aggp_ref[1]
    cnt = cntT_ref[:, 0:1] + cntT_ref[:, 1:2]
    mean = a / jnp.maximum(cnt, 1.0)
    h = (jnp.dot(mean, wlT_ref[...], preferred_element_type=jnp.float32)
         + bl_ref[...]
         + jnp.dot(x_ref[...], wrT_ref[...], preferred_element_type=jnp.float32))
    out_ref[...] = jnp.maximum(h, 0.0)


def _tc2_body(aggp_ref, cntT_ref, h_ref, wlT_ref, wrT_ref, bl_ref, out_ref):
    a = aggp_ref[0] + aggp_ref[1]
    cnt = cntT_ref[:, 0:1] + cntT_ref[:, 1:2]
    mean = a / jnp.maximum(cnt, 1.0)
    h2 = (jnp.dot(mean, wlT_ref[...], preferred_element_type=jnp.float32)
          + bl_ref[...]
          + jnp.dot(h_ref[...], wrT_ref[...], preferred_element_type=jnp.float32))
    nrm = jnp.sqrt(jnp.sum(h2 * h2, axis=1, keepdims=True))
    out_ref[...] = h2 / jnp.maximum(nrm, 1e-12)


def _tc_call(body):
    return pl.pallas_call(
        body,
        grid=(N_NODES // BN,),
        in_specs=[
            pl.BlockSpec((NC, BN, D), lambda i: (0, i, 0)),
            pl.BlockSpec((BN, NC), lambda i: (i, 0)),
            pl.BlockSpec((BN, D), lambda i: (i, 0)),
            pl.BlockSpec((D, D), lambda i: (0, 0)),
            pl.BlockSpec((D, D), lambda i: (0, 0)),
            pl.BlockSpec((1, D), lambda i: (0, 0)),
        ],
        out_specs=pl.BlockSpec((BN, D), lambda i: (i, 0)),
        out_shape=jax.ShapeDtypeStruct((N_NODES, D), jnp.float32),
    )


_tc1 = _tc_call(_tc1_body)
_tc2 = _tc_call(_tc2_body)


def kernel(x, edge_index, Wl1, bl1, Wr1, Wl2, bl2, Wr2):
    ei = edge_index.astype(jnp.int32)
    src3 = ei[0].reshape(NW, NCHUNK, CHUNK)
    dst3 = ei[1].reshape(NW, NCHUNK, CHUNK)

    aggp1, cntp = _segsum(x, src3, dst3)
    cntT = cntp.reshape(NC, CNT_PAD)[:, :N_NODES].T
    h = _tc1(aggp1, cntT, x, Wl1.T, Wr1.T, bl1.reshape(1, D))
    aggp2, _ = _segsum(h, src3, dst3)
    return _tc2(aggp2, cntT, h, Wl2.T, Wr2.T, bl2.reshape(1, D))


# speedup vs baseline: 8.9672x; 1.2798x over previous
import jax
import jax.numpy as jnp
from jax import lax
from jax.experimental import pallas as pl
from jax.experimental.pallas import tpu as pltpu
from jax.experimental.pallas import tpu_sc as plsc

N_NODES = 10000
N_EDGES = 320000
D = 128

NC = 2
NS = 16
NW = NC * NS
CHUNK = 80
HALVES = 2
CPH = 64
EPW = HALVES * CPH * CHUNK
E_PAD = NW * EPW
N_DUMP = 240
AGG_ROWS = N_NODES + N_DUMP
IO_TILES = 10
ROWS_PT = 1024
OUT_PT = N_NODES // IO_TILES
CNT_PAD = 10240
CNT_PT = CNT_PAD // IO_TILES


def _make_segsum(with_cnt):
    def body(x_hbm, src_hbm, dst_hbm, *refs):
        if with_cnt:
            (out_agg, out_cnt, idx_s, idx_d, rows0, rows1, ones_v, zbuf,
             zcbuf, agg_sh, cnt_sh, sem_g) = refs
        else:
            (out_agg, idx_s, idx_d, rows0, rows1, zbuf, agg_sh, sem_g) = refs

        c = lax.axis_index("c")
        s = lax.axis_index("s")
        wid = s * NC + c
        zero16 = jnp.zeros((16,), jnp.float32)

        def zrow(i, carry):
            for k in range(8):
                zbuf[i, pl.ds(16 * k, 16)] = zero16
            return carry
        lax.fori_loop(0, zbuf.shape[0], zrow, 0)

        if with_cnt:
            def zc(i, carry):
                zcbuf[pl.ds(i * 16, 16)] = zero16
                return carry
            lax.fori_loop(0, zcbuf.shape[0] // 16, zc, 0)
            one16 = jnp.ones((16,), jnp.float32)
            for k in range(CHUNK // 16):
                ones_v[pl.ds(16 * k, 16)] = one16

        @pl.when(s < IO_TILES)
        def _():
            def zcopy(k, carry):
                pltpu.sync_copy(zbuf, agg_sh.at[pl.ds(s * ROWS_PT + k * 16, 16)])
                return carry
            lax.fori_loop(0, ROWS_PT // 16, zcopy, 0)
            if with_cnt:
                pltpu.sync_copy(zcbuf, cnt_sh.at[pl.ds(s * CNT_PT, CNT_PT)])

        plsc.subcore_barrier()

        def gather(idx_row, dst_ref):
            pltpu.async_copy(x_hbm.at[idx_row], dst_ref, sem_g)

        def gwait(idx_row, dst_ref):
            pltpu.make_async_copy(x_hbm.at[idx_row], dst_ref, sem_g).wait()

        def scat(j, rows_ref):
            di = idx_d.at[j]
            pltpu.sync_copy(rows_ref, agg_sh.at[di], add=True)
            if with_cnt:
                pltpu.sync_copy(ones_v, cnt_sh.at[di], add=True)

        for h in range(HALVES):
            pltpu.sync_copy(src_hbm.at[wid, h], idx_s)
            pltpu.sync_copy(dst_hbm.at[wid, h], idx_d)
            gather(idx_s.at[0], rows0)

            def pair(i, carry):
                j0 = 2 * i
                gwait(idx_s.at[j0], rows0)
                gather(idx_s.at[j0 + 1], rows1)
                scat(j0, rows0)
                gwait(idx_s.at[j0 + 1], rows1)

                @pl.when(i < CPH // 2 - 1)
                def _():
                    gather(idx_s.at[j0 + 2], rows0)
                scat(j0 + 1, rows1)
                return carry
            lax.fori_loop(0, CPH // 2, pair, 0)

        plsc.subcore_barrier()

        @pl.when(s < IO_TILES)
        def _():
            pltpu.sync_copy(agg_sh.at[pl.ds(s * OUT_PT, OUT_PT)],
                            out_agg.at[c, pl.ds(s * OUT_PT, OUT_PT)])
            if with_cnt:
                pltpu.sync_copy(cnt_sh.at[pl.ds(s * CNT_PT, CNT_PT)],
                                out_cnt.at[pl.ds(c * CNT_PAD + s * CNT_PT, CNT_PT)])

    out_type = [jax.ShapeDtypeStruct((NC, N_NODES, D), jnp.float32)]
    scratch = [
        pltpu.VMEM((CPH, CHUNK), jnp.int32),
        pltpu.VMEM((CPH, CHUNK), jnp.int32),
        pltpu.VMEM((CHUNK, D), jnp.float32),
        pltpu.VMEM((CHUNK, D), jnp.float32),
    ]
    if with_cnt:
        out_type.append(jax.ShapeDtypeStruct((NC * CNT_PAD,), jnp.float32))
        scratch.append(pltpu.VMEM((CHUNK,), jnp.float32))
    scratch.append(pltpu.VMEM((16, D), jnp.float32))
    if with_cnt:
        scratch.append(pltpu.VMEM((CNT_PT,), jnp.float32))
    scratch.append(pltpu.VMEM_SHARED((AGG_ROWS, D), jnp.float32))
    if with_cnt:
        scratch.append(pltpu.VMEM_SHARED((CNT_PAD,), jnp.float32))
    scratch.append(pltpu.SemaphoreType.DMA)

    return pl.kernel(
        body,
        mesh=plsc.VectorSubcoreMesh(core_axis_name="c", subcore_axis_name="s"),
        out_type=out_type,
        scratch_types=scratch,
    )


_segsum_cnt = _make_segsum(True)
_segsum_nocnt = _make_segsum(False)


BN = 1000


def _tc1_body(aggp_ref, cntT_ref, x_ref, wlT_ref, wrT_ref, bl_ref, out_ref):
    a = aggp_ref[0] + aggp_ref[1]
    cnt = cntT_ref[:, 0:1] + cntT_ref[:, 1:2]
    mean = a / jnp.maximum(cnt, 1.0)
    h = (jnp.dot(mean, wlT_ref[...], preferred_element_type=jnp.float32)
         + bl_ref[...]
         + jnp.dot(x_ref[...], wrT_ref[...], preferred_element_type=jnp.float32))
    out_ref[...] = jnp.maximum(h, 0.0)


def _tc2_body(aggp_ref, cntT_ref, h_ref, wlT_ref, wrT_ref, bl_ref, out_ref):
    a = aggp_ref[0] + aggp_ref[1]
    cnt = cntT_ref[:, 0:1] + cntT_ref[:, 1:2]
    mean = a / jnp.maximum(cnt, 1.0)
    h2 = (jnp.dot(mean, wlT_ref[...], preferred_element_type=jnp.float32)
          + bl_ref[...]
          + jnp.dot(h_ref[...], wrT_ref[...], preferred_element_type=jnp.float32))
    nrm = jnp.sqrt(jnp.sum(h2 * h2, axis=1, keepdims=True))
    out_ref[...] = h2 / jnp.maximum(nrm, 1e-12)


def _tc_call(body):
    return pl.pallas_call(
        body,
        grid=(N_NODES // BN,),
        in_specs=[
            pl.BlockSpec((NC, BN, D), lambda i: (0, i, 0)),
            pl.BlockSpec((BN, NC), lambda i: (i, 0)),
            pl.BlockSpec((BN, D), lambda i: (i, 0)),
            pl.BlockSpec((D, D), lambda i: (0, 0)),
            pl.BlockSpec((D, D), lambda i: (0, 0)),
            pl.BlockSpec((1, D), lambda i: (0, 0)),
        ],
        out_specs=pl.BlockSpec((BN, D), lambda i: (i, 0)),
        out_shape=jax.ShapeDtypeStruct((N_NODES, D), jnp.float32),
    )


_tc1 = _tc_call(_tc1_body)
_tc2 = _tc_call(_tc2_body)


def kernel(x, edge_index, Wl1, bl1, Wr1, Wl2, bl2, Wr2):
    ei = edge_index.astype(jnp.int32)
    n_pad = E_PAD - N_EDGES
    pad_iota = jnp.arange(n_pad, dtype=jnp.int32)
    pad_src = (pad_iota * 37) % N_NODES
    pad_dst = N_NODES + pad_iota % N_DUMP
    src4 = jnp.concatenate([ei[0], pad_src]).reshape(NW, HALVES, CPH, CHUNK)
    dst4 = jnp.concatenate([ei[1], pad_dst]).reshape(NW, HALVES, CPH, CHUNK)

    aggp1, cntp = _segsum_cnt(x, src4, dst4)
    cntT = cntp.reshape(NC, CNT_PAD)[:, :N_NODES].T
    h = _tc1(aggp1, cntT, x, Wl1.T, Wr1.T, bl1.reshape(1, D))
    aggp2 = _segsum_nocnt(h, src4, dst4)[0]
    return _tc2(aggp2, cntT, h, Wl2.T, Wr2.T, bl2.reshape(1, D))
